# baseline (speedup 1.0000x reference)
import functools

import jax
import jax.numpy as jnp
from jax import lax
from jax.experimental import pallas as pl
from jax.experimental.pallas import tpu as pltpu
from jax.experimental.pallas import tpu_sc as plsc

N = 10000
E = 320000
D = 128
S = 3000

NC, NS = 2, 16
K = 128

NP = ((N + 1 + 511) // 512) * 512
NPH = NP // 2
RT_H = NPH // NS
ACC_R = NPH + K
DUMP = NPH
ROWS_T = NP // NS
EDIR = 2 * E
CPT = 8 * -(-EDIR // (NS * K * 8))
EPAD = CPT * NS * K
SKROW = 96
SPAD = -(-S // (NS * SKROW)) * (NS * SKROW)
JS = SPAD // (NS * SKROW)

_mesh = plsc.VectorSubcoreMesh(
    core_axis_name="c", subcore_axis_name="s", num_cores=NC, num_subcores=NS)


SB = 24
NSB = -(-CPT // SB)
CAP = CPT * K
FL = 2048
STG = FL + 304
PCH = 2 * K


@functools.partial(
    pl.kernel,
    out_type=[
        jax.ShapeDtypeStruct((2 * NC * NS * CAP,), jnp.int32),
        jax.ShapeDtypeStruct((2 * NC * NS * CAP,), jnp.int32),
        jax.ShapeDtypeStruct((NC * NS * 16,), jnp.int32),
        jax.ShapeDtypeStruct((NC * NS * 16,), jnp.int32),
        jax.ShapeDtypeStruct((NC * NP,), jnp.float32),
        jax.ShapeDtypeStruct((NC * NS * NP,), jnp.float32),
    ],
    mesh=_mesh,
    scratch_types=[
        pltpu.VMEM((CAP,), jnp.int32),
        pltpu.VMEM((CAP,), jnp.int32),
        pltpu.VMEM((STG,), jnp.int32),
        pltpu.VMEM((STG,), jnp.int32),
        pltpu.VMEM((STG,), jnp.int32),
        pltpu.VMEM((STG,), jnp.int32),
        pltpu.VMEM((16,), jnp.int32),
        pltpu.VMEM((NP,), jnp.float32),
        pltpu.VMEM((NS, ROWS_T), jnp.float32),
        pltpu.VMEM((ROWS_T,), jnp.float32),
    ],
    compiler_params=pltpu.CompilerParams(needs_layout_passes=False),
)
def _bin_kernel(src_hbm, dst_hbm, bsrc_hbm, bdst_hbm, cnta_hbm, cntb_hbm,
                deg_hbm, hist_hbm, src_v, dst_v, sta_s, sta_d, stb_s, stb_d,
                cnt_v, hist_v, hbuf_v, red_v):
    c = lax.axis_index("c")
    s = lax.axis_index("s")
    tile = c * NS + s
    pltpu.sync_copy(src_hbm.at[pl.ds(tile * CAP, CAP)], src_v)
    pltpu.sync_copy(dst_hbm.at[pl.ds(tile * CAP, CAP)], dst_v)

    src_inert = jnp.zeros((16,), jnp.int32) + (c * NP + N)
    dst_inert = jnp.arange(16, dtype=jnp.int32) + DUMP
    ones16 = jnp.full((16,), 1.0, jnp.float32)

    def zero(i, carry):
        hist_v[pl.ds(i * 16, 16)] = jnp.zeros((16,), jnp.float32)
        return carry

    lax.fori_loop(0, NP // 16, zero, 0)

    def half_flush(st_s, st_d, out_s, out_d, rbase, fl, off):
        dsto = pl.multiple_of(rbase + fl, FL)
        pltpu.sync_copy(st_s.at[pl.ds(0, FL)], out_s.at[pl.ds(dsto, FL)])
        pltpu.sync_copy(st_d.at[pl.ds(0, FL)], out_d.at[pl.ds(dsto, FL)])
        tail_s = st_s[pl.ds(FL, 16)]
        tail_d = st_d[pl.ds(FL, 16)]
        st_s[pl.ds(0, 16)] = tail_s
        st_d[pl.ds(0, 16)] = tail_d
        return fl + FL, off - FL

    rba = (0 * NC * NS + tile) * CAP
    rbb = (1 * NC * NS + tile) * CAP

    def vbody(i, carry):
        offa, offb, fla, flb = carry
        srcv = src_v[pl.ds(i * 16, 16)]
        dstv = dst_v[pl.ds(i * 16, 16)]
        plsc.addupdate_scatter(hist_v, [dstv], ones16)
        ma = dstv < NPH
        mb = jnp.logical_not(ma)
        cuma = plsc.cumsum(ma.astype(jnp.int32))
        cumb = plsc.cumsum(mb.astype(jnp.int32))
        posa = cuma + (offa - 1)
        posb = cumb + (offb - 1)
        plsc.store_scatter(sta_s, [posa], srcv, mask=ma)
        plsc.store_scatter(sta_d, [posa], dstv, mask=ma)
        plsc.store_scatter(stb_s, [posb], srcv, mask=mb)
        plsc.store_scatter(stb_d, [posb], dstv - NPH, mask=mb)
        na = jnp.max(cuma)
        offa = offa + na
        offb = offb + (16 - na)

        fla, offa = lax.cond(
            offa >= FL,
            lambda: half_flush(sta_s, sta_d, bsrc_hbm, bdst_hbm, rba,
                               fla, offa),
            lambda: (fla, offa))
        flb, offb = lax.cond(
            offb >= FL,
            lambda: half_flush(stb_s, stb_d, bsrc_hbm, bdst_hbm, rbb,
                               flb, offb),
            lambda: (flb, offb))
        return offa, offb, fla, flb

    z = jnp.int32(0)
    offa, offb, fla, flb = lax.fori_loop(0, CAP // 16, vbody, (z, z, z, z))

    def half_epilogue(st_s, st_d, out_s, out_d, rbase, fl, off, cnt_hbm):
        total = fl + off
        target = ((total + PCH) // PCH) * PCH
        lanes = jnp.arange(16, dtype=jnp.int32)
        plsc.store_scatter(st_s, [lanes + off], src_inert)
        plsc.store_scatter(st_d, [lanes + off], dst_inert)
        r16 = ((total + 15) // 16) * 16 - fl
        nfull = (target - fl - r16) // 16

        def padb(k2, carry2):
            po = pl.multiple_of(r16 + k2 * 16, 16)
            st_s[pl.ds(po, 16)] = src_inert
            st_d[pl.ds(po, 16)] = dst_inert
            return carry2

        lax.fori_loop(0, nfull, padb, 0)

        def flushb(k2, carry2):
            so = pl.multiple_of(k2 * PCH, PCH)
            do = pl.multiple_of(rbase + fl + k2 * PCH, PCH)
            pltpu.sync_copy(st_s.at[pl.ds(so, PCH)], out_s.at[pl.ds(do, PCH)])
            pltpu.sync_copy(st_d.at[pl.ds(so, PCH)], out_d.at[pl.ds(do, PCH)])
            return carry2

        lax.fori_loop(0, (target - fl) // PCH, flushb, 0)
        cnt_v[...] = jnp.zeros((16,), jnp.int32) + target // PCH
        pltpu.sync_copy(cnt_v, cnt_hbm.at[pl.ds(tile * 16, 16)])

    half_epilogue(sta_s, sta_d, bsrc_hbm, bdst_hbm, rba, fla, offa, cnta_hbm)
    half_epilogue(stb_s, stb_d, bsrc_hbm, bdst_hbm, rbb, flb, offb, cntb_hbm)

    pltpu.sync_copy(hist_v, hist_hbm.at[pl.ds(tile * NP, NP)])
    plsc.subcore_barrier()
    for t in range(NS):
        pltpu.sync_copy(
            hist_hbm.at[pl.ds((c * NS + t) * NP + s * ROWS_T, ROWS_T)],
            hbuf_v.at[t])

    def dred(j, carry):
        acc = hbuf_v[0, pl.ds(j * 16, 16)]
        for t in range(1, NS):
            acc = acc + hbuf_v[t, pl.ds(j * 16, 16)]
        red_v[pl.ds(j * 16, 16)] = acc
        return carry

    lax.fori_loop(0, ROWS_T // 16, dred, 0)
    pltpu.sync_copy(red_v, deg_hbm.at[pl.ds(c * NP + s * ROWS_T, ROWS_T)])


@functools.partial(
    pl.kernel,
    out_type=jax.ShapeDtypeStruct((NC * NP, D), jnp.float32),
    mesh=_mesh,
    scratch_types=[
        pltpu.VMEM((2, SB, K), jnp.int32),
        pltpu.VMEM((2, SB, K), jnp.int32),
        pltpu.VMEM((2, K, D), jnp.float32),
        pltpu.VMEM((16,), jnp.int32),
        pltpu.VMEM_SHARED((ACC_R, D), jnp.float32),
        pltpu.SemaphoreType.DMA,
        pltpu.SemaphoreType.DMA,
        pltpu.SemaphoreType.DMA,
        pltpu.SemaphoreType.DMA,
    ],
)
def _edge_kernel(hw_hbm, src_hbm, dst_hbm, cnta_hbm, cntb_hbm, acc_hbm,
                 srcr_v, dstr_v, rows_v, cnt_v, acc_sp, sem0, sem1, semis,
                 semid):
    c = lax.axis_index("c")
    s = lax.axis_index("s")
    tile = c * NS + s

    for h in range(2):
        cnt_hbm = (cnta_hbm, cntb_hbm)[h]
        base = h * (NC * NS * CPT) + tile * CPT
        pltpu.sync_copy(cnt_hbm.at[pl.ds(tile * 16, 16)], cnt_v)
        npairs = cnt_v[...][0]
        nsb = (2 * npairs + SB - 1) // SB
        pltpu.sync_copy(src_hbm.at[pl.ds(base, SB)], srcr_v.at[0])
        pltpu.sync_copy(dst_hbm.at[pl.ds(base, SB)], dstr_v.at[0])
        pltpu.sync_copy(
            hw_hbm.at[pl.ds(c * NP + h * NPH + s * RT_H, RT_H)],
            acc_sp.at[pl.ds(s * RT_H, RT_H)])
        pltpu.async_copy(hw_hbm.at[srcr_v.at[0, 0]], rows_v.at[0], sem0)
        plsc.subcore_barrier()

        def outer(j, carry):
            p = lax.rem(j, 2)

            @pl.when(j + 1 < nsb)
            def _():
                pltpu.async_copy(src_hbm.at[pl.ds(base + (j + 1) * SB, SB)],
                                 srcr_v.at[1 - p], semis)
                pltpu.async_copy(dst_hbm.at[pl.ds(base + (j + 1) * SB, SB)],
                                 dstr_v.at[1 - p], semid)

            pr = jnp.minimum(SB // 2, npairs - j * (SB // 2))

            def pair(q, carry2):
                q0 = 2 * q
                cp1 = pltpu.async_copy(
                    hw_hbm.at[srcr_v.at[p, q0 + 1]], rows_v.at[1], sem1)
                pltpu.make_async_copy(
                    hw_hbm.at[srcr_v.at[p, q0]], rows_v.at[0], sem0).wait()
                pltpu.sync_copy(rows_v.at[0], acc_sp.at[dstr_v.at[p, q0]],
                                add=True)

                @pl.when(q + 1 < pr)
                def _():
                    pltpu.async_copy(
                        hw_hbm.at[srcr_v.at[p, q0 + 2]], rows_v.at[0], sem0)

                cp1.wait()
                pltpu.sync_copy(rows_v.at[1],
                                acc_sp.at[dstr_v.at[p, q0 + 1]], add=True)
                return carry2

            lax.fori_loop(0, pr, pair, 0)

            @pl.when(j + 1 < nsb)
            def _():
                pltpu.make_async_copy(src_hbm.at[pl.ds(base, SB)],
                                      srcr_v.at[1 - p], semis).wait()
                pltpu.make_async_copy(dst_hbm.at[pl.ds(base, SB)],
                                      dstr_v.at[1 - p], semid).wait()
                pltpu.async_copy(hw_hbm.at[srcr_v.at[1 - p, 0]],
                                 rows_v.at[0], sem0)
            return carry

        lax.fori_loop(0, nsb, outer, 0)
        plsc.subcore_barrier()
        pltpu.sync_copy(
            acc_sp.at[pl.ds(s * RT_H, RT_H)],
            acc_hbm.at[pl.ds(c * NP + h * NPH + s * RT_H, RT_H)])


@functools.partial(
    pl.kernel,
    out_type=jax.ShapeDtypeStruct((NC * SPAD, D), jnp.float32),
    mesh=_mesh,
    scratch_types=[
        pltpu.VMEM((JS * SKROW,), jnp.int32),
        pltpu.VMEM((SKROW, D), jnp.float32),
        pltpu.SemaphoreType.DMA,
    ],
)
def _seed_kernel(h_hbm, seed_hbm, out_hbm, idx_v, rows_v, sem):
    c = lax.axis_index("c")
    s = lax.axis_index("s")
    pltpu.sync_copy(
        seed_hbm.at[pl.ds(c * SPAD + s * (JS * SKROW), JS * SKROW)], idx_v)
    for j in range(JS):
        pltpu.async_copy(h_hbm.at[idx_v.at[pl.ds(j * SKROW, SKROW)]],
                         rows_v, sem).wait()
        pltpu.sync_copy(
            rows_v,
            out_hbm.at[pl.ds(c * SPAD + s * (JS * SKROW) + j * SKROW, SKROW)])


R = 512
G = (NC * NP) // R
BPG = NP // R
BPH = NPH // R


def _dinv(deg_blk):
    return lax.rsqrt(jnp.maximum(deg_blk + 1.0, 1.0))


def _mm_body(emb_ref, deg_ref, w_ref, out_ref):
    dinv = _dinv(deg_ref[...])
    out_ref[...] = jnp.dot(emb_ref[...], w_ref[...],
                           preferred_element_type=jnp.float32) * dinv


_mm = pl.pallas_call(
    _mm_body,
    grid=(G,),
    in_specs=[
        pl.BlockSpec((R, D), lambda i: (i, 0)),
        pl.BlockSpec((R, 1), lambda i: (i, 0)),
        pl.BlockSpec((D, D), lambda i: (0, 0)),
    ],
    out_specs=pl.BlockSpec((R, D), lambda i: (i, 0)),
    out_shape=jax.ShapeDtypeStruct((NC * NP, D), jnp.float32),
)


def _layer_body(acc_ref, emb_ref, deg_ref, w_ref, h1_ref, hw_ref):
    dinv = _dinv(deg_ref[...])
    h1 = jnp.maximum(acc_ref[...] * dinv, 0.0) + emb_ref[...]
    h1_ref[...] = h1
    hw_ref[...] = jnp.dot(h1, w_ref[...],
                          preferred_element_type=jnp.float32) * dinv


_layer = pl.pallas_call(
    _layer_body,
    grid=(G,),
    in_specs=[
        pl.BlockSpec((R, D), lambda i: (i, 0)),
        pl.BlockSpec((R, D), lambda i: (i, 0)),
        pl.BlockSpec((R, 1), lambda i: (i, 0)),
        pl.BlockSpec((D, D), lambda i: (0, 0)),
    ],
    out_specs=[
        pl.BlockSpec((R, D), lambda i: (i, 0)),
        pl.BlockSpec((R, D), lambda i: (i, 0)),
    ],
    out_shape=[
        jax.ShapeDtypeStruct((NC * NP, D), jnp.float32),
        jax.ShapeDtypeStruct((NC * NP, D), jnp.float32),
    ],
)


def kernel(sr_ent_seeds, tg_ent_seeds, triples_sr, triples_tg,
           embedding_sr, embedding_tg, edges_sr, edges_tg, W0, W1):
    del triples_sr, triples_tg

    emb = jnp.stack([embedding_sr, embedding_tg])
    emb = jnp.pad(emb, ((0, 0), (0, NP - N), (0, 0))).reshape(NC * NP, D)

    spread = jnp.arange(EPAD - EDIR, dtype=jnp.int32) % (NP - N)

    def _dirs(e, g):
        s0, d0 = e[:, 0], e[:, 1]
        src = jnp.concatenate([s0 + g * NP, d0 + g * NP,
                               g * NP + N + spread])
        dst = jnp.concatenate([d0, s0, N + spread])
        return src, dst

    ssr, dsr = _dirs(edges_sr, 0)
    stg, dtg = _dirs(edges_tg, 1)
    src1d = jnp.concatenate([ssr, stg])
    dst1d = jnp.concatenate([dsr, dtg])
    seeds = jnp.stack([sr_ent_seeds, tg_ent_seeds])
    seeds = jnp.pad(seeds, ((0, 0), (0, SPAD - S)))
    seeds = (seeds + jnp.array([[0], [NP]], jnp.int32)).reshape(-1)

    bsrc, bdst, cnta, cntb, deg, _ = _bin_kernel(src1d, dst1d)
    deg = deg.reshape(NC * NP, 1)
    bsrc2d = bsrc.reshape(-1, K)
    bdst2d = bdst.reshape(-1, K)
    hw0 = _mm(emb, deg, W0)
    acc0 = _edge_kernel(hw0, bsrc2d, bdst2d, cnta, cntb)
    h1, hw1 = _layer(acc0, emb, deg, W1)
    acc1 = _edge_kernel(hw1, bsrc2d, bdst2d, cnta, cntb)
    h2, _ = _layer(acc1, h1, deg, W1)
    seed_out = _seed_kernel(h2, seeds)

    h2r = h2.reshape(NC, NP, D)
    so = seed_out.reshape(NC, SPAD, D)
    return (so[0, :S], so[1, :S], h2r[0, :N], h2r[1, :N])


# speedup vs baseline: 1.0092x; 1.0092x over previous
import functools

import jax
import jax.numpy as jnp
from jax import lax
from jax.experimental import pallas as pl
from jax.experimental.pallas import tpu as pltpu
from jax.experimental.pallas import tpu_sc as plsc

N = 10000
E = 320000
D = 128
S = 3000

NC, NS = 2, 16
K = 128

NP = ((N + 1 + 511) // 512) * 512
NPH = NP // 2
RT_H = NPH // NS
ACC_R = NPH + K
DUMP = NPH
ROWS_T = NP // NS
EDIR = 2 * E
CPT = 8 * -(-EDIR // (NS * K * 8))
EPAD = CPT * NS * K
SKROW = 96
SPAD = -(-S // (NS * SKROW)) * (NS * SKROW)
JS = SPAD // (NS * SKROW)

_mesh = plsc.VectorSubcoreMesh(
    core_axis_name="c", subcore_axis_name="s", num_cores=NC, num_subcores=NS)


SB = 24
NSB = -(-CPT // SB)
CAP = CPT * K
FL = 2048
STG = FL + 304
PCH = 2 * K


@functools.partial(
    pl.kernel,
    out_type=[
        jax.ShapeDtypeStruct((2 * NC * NS * CAP + SB * K,), jnp.int32),
        jax.ShapeDtypeStruct((2 * NC * NS * CAP + SB * K,), jnp.int32),
        jax.ShapeDtypeStruct((NC * NS * 16,), jnp.int32),
        jax.ShapeDtypeStruct((NC * NS * 16,), jnp.int32),
        jax.ShapeDtypeStruct((NC * NP,), jnp.float32),
        jax.ShapeDtypeStruct((NC * NS * NP,), jnp.float32),
    ],
    mesh=_mesh,
    scratch_types=[
        pltpu.VMEM((CAP,), jnp.int32),
        pltpu.VMEM((CAP,), jnp.int32),
        pltpu.VMEM((STG,), jnp.int32),
        pltpu.VMEM((STG,), jnp.int32),
        pltpu.VMEM((STG,), jnp.int32),
        pltpu.VMEM((STG,), jnp.int32),
        pltpu.VMEM((16,), jnp.int32),
        pltpu.VMEM((NP,), jnp.float32),
        pltpu.VMEM((NS, ROWS_T), jnp.float32),
        pltpu.VMEM((ROWS_T,), jnp.float32),
    ],
    compiler_params=pltpu.CompilerParams(needs_layout_passes=False),
)
def _bin_kernel(src_hbm, dst_hbm, bsrc_hbm, bdst_hbm, cnta_hbm, cntb_hbm,
                deg_hbm, hist_hbm, src_v, dst_v, sta_s, sta_d, stb_s, stb_d,
                cnt_v, hist_v, hbuf_v, red_v):
    c = lax.axis_index("c")
    s = lax.axis_index("s")
    tile = c * NS + s
    pltpu.sync_copy(src_hbm.at[pl.ds(tile * CAP, CAP)], src_v)
    pltpu.sync_copy(dst_hbm.at[pl.ds(tile * CAP, CAP)], dst_v)

    src_inert = jnp.zeros((16,), jnp.int32) + (c * NP + N)
    dst_inert = jnp.arange(16, dtype=jnp.int32) + DUMP
    ones16 = jnp.full((16,), 1.0, jnp.float32)

    def zero(i, carry):
        hist_v[pl.ds(i * 16, 16)] = jnp.zeros((16,), jnp.float32)
        return carry

    lax.fori_loop(0, NP // 16, zero, 0)

    def half_flush(st_s, st_d, out_s, out_d, rbase, fl, off):
        dsto = pl.multiple_of(rbase + fl, FL)
        pltpu.sync_copy(st_s.at[pl.ds(0, FL)], out_s.at[pl.ds(dsto, FL)])
        pltpu.sync_copy(st_d.at[pl.ds(0, FL)], out_d.at[pl.ds(dsto, FL)])
        tail_s = st_s[pl.ds(FL, 16)]
        tail_d = st_d[pl.ds(FL, 16)]
        st_s[pl.ds(0, 16)] = tail_s
        st_d[pl.ds(0, 16)] = tail_d
        return fl + FL, off - FL

    rba = (0 * NC * NS + tile) * CAP
    rbb = (1 * NC * NS + tile) * CAP

    def vbody(i, carry):
        offa, offb, fla, flb = carry
        srcv = src_v[pl.ds(i * 16, 16)]
        dstv = dst_v[pl.ds(i * 16, 16)]
        plsc.addupdate_scatter(hist_v, [dstv], ones16)
        ma = dstv < NPH
        mb = jnp.logical_not(ma)
        cuma = plsc.cumsum(ma.astype(jnp.int32))
        cumb = plsc.cumsum(mb.astype(jnp.int32))
        posa = cuma + (offa - 1)
        posb = cumb + (offb - 1)
        plsc.store_scatter(sta_s, [posa], srcv, mask=ma)
        plsc.store_scatter(sta_d, [posa], dstv, mask=ma)
        plsc.store_scatter(stb_s, [posb], srcv, mask=mb)
        plsc.store_scatter(stb_d, [posb], dstv - NPH, mask=mb)
        na = jnp.max(cuma)
        offa = offa + na
        offb = offb + (16 - na)

        fla, offa = lax.cond(
            offa >= FL,
            lambda: half_flush(sta_s, sta_d, bsrc_hbm, bdst_hbm, rba,
                               fla, offa),
            lambda: (fla, offa))
        flb, offb = lax.cond(
            offb >= FL,
            lambda: half_flush(stb_s, stb_d, bsrc_hbm, bdst_hbm, rbb,
                               flb, offb),
            lambda: (flb, offb))
        return offa, offb, fla, flb

    z = jnp.int32(0)
    offa, offb, fla, flb = lax.fori_loop(0, CAP // 16, vbody, (z, z, z, z))

    def half_epilogue(st_s, st_d, out_s, out_d, rbase, fl, off, cnt_hbm):
        total = fl + off
        target = ((total + PCH) // PCH) * PCH
        lanes = jnp.arange(16, dtype=jnp.int32)
        plsc.store_scatter(st_s, [lanes + off], src_inert)
        plsc.store_scatter(st_d, [lanes + off], dst_inert)
        r16 = ((total + 15) // 16) * 16 - fl
        nfull = (target - fl - r16) // 16

        def padb(k2, carry2):
            po = pl.multiple_of(r16 + k2 * 16, 16)
            st_s[pl.ds(po, 16)] = src_inert
            st_d[pl.ds(po, 16)] = dst_inert
            return carry2

        lax.fori_loop(0, nfull, padb, 0)

        def flushb(k2, carry2):
            so = pl.multiple_of(k2 * PCH, PCH)
            do = pl.multiple_of(rbase + fl + k2 * PCH, PCH)
            pltpu.sync_copy(st_s.at[pl.ds(so, PCH)], out_s.at[pl.ds(do, PCH)])
            pltpu.sync_copy(st_d.at[pl.ds(so, PCH)], out_d.at[pl.ds(do, PCH)])
            return carry2

        lax.fori_loop(0, (target - fl) // PCH, flushb, 0)
        cnt_v[...] = jnp.zeros((16,), jnp.int32) + target // PCH
        pltpu.sync_copy(cnt_v, cnt_hbm.at[pl.ds(tile * 16, 16)])

    half_epilogue(sta_s, sta_d, bsrc_hbm, bdst_hbm, rba, fla, offa, cnta_hbm)
    half_epilogue(stb_s, stb_d, bsrc_hbm, bdst_hbm, rbb, flb, offb, cntb_hbm)

    pltpu.sync_copy(hist_v, hist_hbm.at[pl.ds(tile * NP, NP)])
    plsc.subcore_barrier()
    for t in range(NS):
        pltpu.sync_copy(
            hist_hbm.at[pl.ds((c * NS + t) * NP + s * ROWS_T, ROWS_T)],
            hbuf_v.at[t])

    def dred(j, carry):
        acc = hbuf_v[0, pl.ds(j * 16, 16)]
        for t in range(1, NS):
            acc = acc + hbuf_v[t, pl.ds(j * 16, 16)]
        red_v[pl.ds(j * 16, 16)] = acc
        return carry

    lax.fori_loop(0, ROWS_T // 16, dred, 0)
    pltpu.sync_copy(red_v, deg_hbm.at[pl.ds(c * NP + s * ROWS_T, ROWS_T)])


@functools.partial(
    pl.kernel,
    out_type=jax.ShapeDtypeStruct((NC * NP, D), jnp.float32),
    mesh=_mesh,
    scratch_types=[
        pltpu.VMEM((2, SB, K), jnp.int32),
        pltpu.VMEM((2, SB, K), jnp.int32),
        pltpu.VMEM((2, K, D), jnp.float32),
        pltpu.VMEM((16,), jnp.int32),
        pltpu.VMEM_SHARED((ACC_R, D), jnp.float32),
        pltpu.SemaphoreType.DMA,
        pltpu.SemaphoreType.DMA,
        pltpu.SemaphoreType.DMA,
        pltpu.SemaphoreType.DMA,
    ],
)
def _edge_kernel(hw_hbm, src_hbm, dst_hbm, cnta_hbm, cntb_hbm, acc_hbm,
                 srcr_v, dstr_v, rows_v, cnt_v, acc_sp, sem0, sem1, semis,
                 semid):
    c = lax.axis_index("c")
    s = lax.axis_index("s")
    tile = c * NS + s

    for h in range(2):
        cnt_hbm = (cnta_hbm, cntb_hbm)[h]
        base = h * (NC * NS * CPT) + tile * CPT
        pltpu.sync_copy(cnt_hbm.at[pl.ds(tile * 16, 16)], cnt_v)
        npairs = cnt_v[...][0]
        nsb = (2 * npairs + SB - 1) // SB
        pltpu.sync_copy(src_hbm.at[pl.ds(base, SB)], srcr_v.at[0])
        pltpu.sync_copy(dst_hbm.at[pl.ds(base, SB)], dstr_v.at[0])
        pltpu.sync_copy(
            hw_hbm.at[pl.ds(c * NP + h * NPH + s * RT_H, RT_H)],
            acc_sp.at[pl.ds(s * RT_H, RT_H)])
        pltpu.async_copy(hw_hbm.at[srcr_v.at[0, 0]], rows_v.at[0], sem0)
        plsc.subcore_barrier()

        def outer(j, carry):
            p = lax.rem(j, 2)

            @pl.when(j + 1 < nsb)
            def _():
                pltpu.async_copy(src_hbm.at[pl.ds(base + (j + 1) * SB, SB)],
                                 srcr_v.at[1 - p], semis)
                pltpu.async_copy(dst_hbm.at[pl.ds(base + (j + 1) * SB, SB)],
                                 dstr_v.at[1 - p], semid)

            pr = jnp.minimum(SB // 2, npairs - j * (SB // 2))

            def pair(q, carry2):
                q0 = 2 * q
                cp1 = pltpu.async_copy(
                    hw_hbm.at[srcr_v.at[p, q0 + 1]], rows_v.at[1], sem1)
                pltpu.make_async_copy(
                    hw_hbm.at[srcr_v.at[p, q0]], rows_v.at[0], sem0).wait()
                pltpu.sync_copy(rows_v.at[0], acc_sp.at[dstr_v.at[p, q0]],
                                add=True)

                @pl.when(q + 1 < pr)
                def _():
                    pltpu.async_copy(
                        hw_hbm.at[srcr_v.at[p, q0 + 2]], rows_v.at[0], sem0)

                cp1.wait()
                pltpu.sync_copy(rows_v.at[1],
                                acc_sp.at[dstr_v.at[p, q0 + 1]], add=True)
                return carry2

            lax.fori_loop(0, pr, pair, 0)

            @pl.when(j + 1 < nsb)
            def _():
                pltpu.make_async_copy(src_hbm.at[pl.ds(base, SB)],
                                      srcr_v.at[1 - p], semis).wait()
                pltpu.make_async_copy(dst_hbm.at[pl.ds(base, SB)],
                                      dstr_v.at[1 - p], semid).wait()
                pltpu.async_copy(hw_hbm.at[srcr_v.at[1 - p, 0]],
                                 rows_v.at[0], sem0)
            return carry

        lax.fori_loop(0, nsb, outer, 0)
        plsc.subcore_barrier()
        pltpu.sync_copy(
            acc_sp.at[pl.ds(s * RT_H, RT_H)],
            acc_hbm.at[pl.ds(c * NP + h * NPH + s * RT_H, RT_H)])


@functools.partial(
    pl.kernel,
    out_type=jax.ShapeDtypeStruct((NC * SPAD, D), jnp.float32),
    mesh=_mesh,
    scratch_types=[
        pltpu.VMEM((JS * SKROW,), jnp.int32),
        pltpu.VMEM((SKROW, D), jnp.float32),
        pltpu.SemaphoreType.DMA,
    ],
)
def _seed_kernel(h_hbm, seed_hbm, out_hbm, idx_v, rows_v, sem):
    c = lax.axis_index("c")
    s = lax.axis_index("s")
    pltpu.sync_copy(
        seed_hbm.at[pl.ds(c * SPAD + s * (JS * SKROW), JS * SKROW)], idx_v)
    for j in range(JS):
        pltpu.async_copy(h_hbm.at[idx_v.at[pl.ds(j * SKROW, SKROW)]],
                         rows_v, sem).wait()
        pltpu.sync_copy(
            rows_v,
            out_hbm.at[pl.ds(c * SPAD + s * (JS * SKROW) + j * SKROW, SKROW)])


R = 512
G = (NC * NP) // R
BPG = NP // R
BPH = NPH // R


def _dinv(deg_blk):
    return lax.rsqrt(jnp.maximum(deg_blk + 1.0, 1.0))


def _mm_body(emb_ref, deg_ref, w_ref, out_ref):
    dinv = _dinv(deg_ref[...])
    out_ref[...] = jnp.dot(emb_ref[...], w_ref[...],
                           preferred_element_type=jnp.float32) * dinv


_mm = pl.pallas_call(
    _mm_body,
    grid=(G,),
    in_specs=[
        pl.BlockSpec((R, D), lambda i: (i, 0)),
        pl.BlockSpec((R, 1), lambda i: (i, 0)),
        pl.BlockSpec((D, D), lambda i: (0, 0)),
    ],
    out_specs=pl.BlockSpec((R, D), lambda i: (i, 0)),
    out_shape=jax.ShapeDtypeStruct((NC * NP, D), jnp.float32),
)


def _layer_body(acc_ref, emb_ref, deg_ref, w_ref, h1_ref, hw_ref):
    dinv = _dinv(deg_ref[...])
    h1 = jnp.maximum(acc_ref[...] * dinv, 0.0) + emb_ref[...]
    h1_ref[...] = h1
    hw_ref[...] = jnp.dot(h1, w_ref[...],
                          preferred_element_type=jnp.float32) * dinv


_layer = pl.pallas_call(
    _layer_body,
    grid=(G,),
    in_specs=[
        pl.BlockSpec((R, D), lambda i: (i, 0)),
        pl.BlockSpec((R, D), lambda i: (i, 0)),
        pl.BlockSpec((R, 1), lambda i: (i, 0)),
        pl.BlockSpec((D, D), lambda i: (0, 0)),
    ],
    out_specs=[
        pl.BlockSpec((R, D), lambda i: (i, 0)),
        pl.BlockSpec((R, D), lambda i: (i, 0)),
    ],
    out_shape=[
        jax.ShapeDtypeStruct((NC * NP, D), jnp.float32),
        jax.ShapeDtypeStruct((NC * NP, D), jnp.float32),
    ],
)


def _final_body(acc_ref, h1_ref, deg_ref, out_ref):
    dinv = _dinv(deg_ref[...])
    out_ref[...] = jnp.maximum(acc_ref[...] * dinv, 0.0) + h1_ref[...]


_final = pl.pallas_call(
    _final_body,
    grid=(G,),
    in_specs=[
        pl.BlockSpec((R, D), lambda i: (i, 0)),
        pl.BlockSpec((R, D), lambda i: (i, 0)),
        pl.BlockSpec((R, 1), lambda i: (i, 0)),
    ],
    out_specs=pl.BlockSpec((R, D), lambda i: (i, 0)),
    out_shape=jax.ShapeDtypeStruct((NC * NP, D), jnp.float32),
)


def kernel(sr_ent_seeds, tg_ent_seeds, triples_sr, triples_tg,
           embedding_sr, embedding_tg, edges_sr, edges_tg, W0, W1):
    del triples_sr, triples_tg

    emb = jnp.stack([embedding_sr, embedding_tg])
    emb = jnp.pad(emb, ((0, 0), (0, NP - N), (0, 0))).reshape(NC * NP, D)

    spread = jnp.arange(EPAD - EDIR, dtype=jnp.int32) % (NP - N)

    def _dirs(e, g):
        s0, d0 = e[:, 0], e[:, 1]
        src = jnp.concatenate([s0 + g * NP, d0 + g * NP,
                               g * NP + N + spread])
        dst = jnp.concatenate([d0, s0, N + spread])
        return src, dst

    ssr, dsr = _dirs(edges_sr, 0)
    stg, dtg = _dirs(edges_tg, 1)
    src1d = jnp.concatenate([ssr, stg])
    dst1d = jnp.concatenate([dsr, dtg])
    seeds = jnp.stack([sr_ent_seeds, tg_ent_seeds])
    seeds = jnp.pad(seeds, ((0, 0), (0, SPAD - S)))
    seeds = (seeds + jnp.array([[0], [NP]], jnp.int32)).reshape(-1)

    bsrc, bdst, cnta, cntb, deg, _ = _bin_kernel(src1d, dst1d)
    deg = deg.reshape(NC * NP, 1)
    bsrc2d = bsrc.reshape(-1, K)
    bdst2d = bdst.reshape(-1, K)
    hw0 = _mm(emb, deg, W0)
    acc0 = _edge_kernel(hw0, bsrc2d, bdst2d, cnta, cntb)
    h1, hw1 = _layer(acc0, emb, deg, W1)
    acc1 = _edge_kernel(hw1, bsrc2d, bdst2d, cnta, cntb)
    h2 = _final(acc1, h1, deg)
    seed_out = _seed_kernel(h2, seeds)

    h2r = h2.reshape(NC, NP, D)
    so = seed_out.reshape(NC, SPAD, D)
    return (so[0, :S], so[1, :S], h2r[0, :N], h2r[1, :N])
